# Initial kernel scaffold; baseline (speedup 1.0000x reference)
import jax
import jax.numpy as jnp
from jax.experimental import pallas as pl


def kernel(x, seq, pause, edge_index, fc_W, fc_b, enc_W, enc_b, lin_l_W, lin_l_b, lin_r_W):
    raise NotImplementedError("write your pallas kernel here")


# speedup vs baseline: 11.8838x; 11.8838x over previous
import dataclasses
import functools

import jax
import jax.numpy as jnp
from jax import lax
from jax.experimental import pallas as pl
from jax.experimental.pallas import tpu as pltpu
from jax.experimental.pallas import tpu_sc as plsc

_N = 10000
_E = 320000
_SEQ_DIM = 9216
_NP = 10240
_BN = 256
_GRID = _NP // _BN
_NSUB = 16
_EPT = _E // _NSUB
_STR = _NP // _NSUB
_L = 16


def _gelu(t):
    return 0.5 * t * (1.0 + lax.erf(t * 0.7071067811865476))


def _dense_body(x_ref, pause_ref, seq_ref, encwt_ref, params_ref, sl_ref, srb_ref):
    enc = jnp.dot(seq_ref[...], encwt_ref[...], preferred_element_type=jnp.float32)
    fcw = params_ref[0, :32]
    fcb = params_ref[1, :32]
    encb = params_ref[2, :32]
    wl = params_ref[3, :64]
    wr = params_ref[4, :64]
    linlb = params_ref[5, 0]
    g = _gelu(enc + encb[None, :])
    h = x_ref[...][:, None] * fcw[None, :] + fcb[None, :] + g
    hp = pause_ref[...][:, None] * fcw[None, :] + fcb[None, :]
    fh = _gelu(h)
    fhp = _gelu(hp)
    sl_ref[...] = jnp.sum(fhp * wl[None, :32], axis=1) + jnp.sum(fh * wl[None, 32:], axis=1)
    srb_ref[...] = (jnp.sum(fhp * wr[None, :32], axis=1)
                    + jnp.sum(fh * wr[None, 32:], axis=1) + linlb)


def _dense(x, pause, seq, encwt, params):
    return pl.pallas_call(
        _dense_body,
        grid=(_GRID,),
        in_specs=[
            pl.BlockSpec((_BN,), lambda i: (i,)),
            pl.BlockSpec((_BN,), lambda i: (i,)),
            pl.BlockSpec((_BN, _SEQ_DIM), lambda i: (i, 0)),
            pl.BlockSpec((_SEQ_DIM, 32), lambda i: (0, 0)),
            pl.BlockSpec((8, 128), lambda i: (0, 0)),
        ],
        out_specs=[pl.BlockSpec((_BN,), lambda i: (i,)),
                   pl.BlockSpec((_BN,), lambda i: (i,))],
        out_shape=[jax.ShapeDtypeStruct((_NP,), jnp.float32),
                   jax.ShapeDtypeStruct((_NP,), jnp.float32)],
    )(x, pause, seq, encwt, params)


def _sc_compiler_params():
    cp = pltpu.CompilerParams()
    if "needs_layout_passes" in pltpu.CompilerParams.__dataclass_fields__:
        cp = dataclasses.replace(cp, needs_layout_passes=False)
    return cp


def _edge(sl, srb, src, dst):
    mesh = plsc.VectorSubcoreMesh(core_axis_name="c", subcore_axis_name="s")

    @functools.partial(
        pl.kernel,
        mesh=mesh,
        compiler_params=_sc_compiler_params(),
        out_type=jax.ShapeDtypeStruct((_NP,), jnp.float32),
        scratch_types=[
            pltpu.VMEM((_NP,), jnp.float32),
            pltpu.VMEM((_NP,), jnp.float32),
            pltpu.VMEM((_NP,), jnp.float32),
            pltpu.VMEM((_EPT,), jnp.int32),
            pltpu.VMEM((_EPT,), jnp.int32),
            pltpu.VMEM((_STR,), jnp.float32),
            pltpu.VMEM((_STR,), jnp.float32),
            pltpu.VMEM((_STR,), jnp.float32),
            pltpu.VMEM((_STR,), jnp.float32),
            pltpu.VMEM_SHARED((_NSUB, _NP), jnp.float32),
            pltpu.VMEM_SHARED((_NSUB, _NP), jnp.float32),
        ],
    )
    def k(sl_hbm, srb_hbm, src_hbm, dst_hbm, out_hbm,
          table, acc, cnt, srcb, dstb, sacc, scnt, stmp, ssrb, acc_all, cnt_all):
        cid = lax.axis_index("c")
        sid = lax.axis_index("s")

        @pl.when(cid == 0)
        def _():
            zv = jnp.zeros((_L,), jnp.float32)
            ones = jnp.ones((_L,), jnp.float32)
            pltpu.sync_copy(sl_hbm, table)
            pltpu.sync_copy(src_hbm.at[pl.ds(sid * _EPT, _EPT)], srcb)
            pltpu.sync_copy(dst_hbm.at[pl.ds(sid * _EPT, _EPT)], dstb)

            @pl.loop(0, _NP, step=_L)
            def _(i):
                acc[pl.ds(i, _L)] = zv
                cnt[pl.ds(i, _L)] = zv

            @pl.loop(0, _EPT, step=_L)
            def _(i):
                sv = srcb[pl.ds(i, _L)]
                dv = dstb[pl.ds(i, _L)]
                vals = plsc.load_gather(table, [sv])
                plsc.addupdate_scatter(acc, [dv], vals)
                plsc.addupdate_scatter(cnt, [dv], ones)

            pltpu.sync_copy(acc, acc_all.at[sid])
            pltpu.sync_copy(cnt, cnt_all.at[sid])
            plsc.subcore_barrier()

            base = sid * _STR
            pltpu.sync_copy(acc_all.at[0, pl.ds(base, _STR)], sacc)
            pltpu.sync_copy(cnt_all.at[0, pl.ds(base, _STR)], scnt)

            @pl.loop(1, _NSUB)
            def _(j):
                pltpu.sync_copy(acc_all.at[j, pl.ds(base, _STR)], stmp)

                @pl.loop(0, _STR, step=_L)
                def _(k2):
                    sacc[pl.ds(k2, _L)] = sacc[pl.ds(k2, _L)] + stmp[pl.ds(k2, _L)]

                pltpu.sync_copy(cnt_all.at[j, pl.ds(base, _STR)], stmp)

                @pl.loop(0, _STR, step=_L)
                def _(k2):
                    scnt[pl.ds(k2, _L)] = scnt[pl.ds(k2, _L)] + stmp[pl.ds(k2, _L)]

            pltpu.sync_copy(srb_hbm.at[pl.ds(base, _STR)], ssrb)

            @pl.loop(0, _STR, step=_L)
            def _(k2):
                mean = sacc[pl.ds(k2, _L)] / jnp.maximum(scnt[pl.ds(k2, _L)], 1.0)
                sacc[pl.ds(k2, _L)] = mean + ssrb[pl.ds(k2, _L)]

            pltpu.sync_copy(sacc, out_hbm.at[pl.ds(base, _STR)])

    return k(sl, srb, src, dst)


def kernel(x, seq, pause, edge_index, fc_W, fc_b, enc_W, enc_b, lin_l_W, lin_l_b, lin_r_W):
    f32 = jnp.float32
    src = edge_index[0].astype(jnp.int32)
    dst = edge_index[1].astype(jnp.int32)
    encwt = enc_W.T.astype(f32)
    params = jnp.zeros((8, 128), f32)
    params = params.at[0, :32].set(fc_W[:, 0])
    params = params.at[1, :32].set(fc_b)
    params = params.at[2, :32].set(enc_b)
    params = params.at[3, :64].set(lin_l_W[0])
    params = params.at[4, :64].set(lin_r_W[0])
    params = params.at[5, 0].set(lin_l_b[0])
    sl, srb = _dense(x, pause, seq, encwt, params)
    out = _edge(sl, srb, src, dst)
    return out[:_N].reshape(_N, 1)


# speedup vs baseline: 11.9250x; 1.0035x over previous
import dataclasses
import functools

import jax
import jax.numpy as jnp
from jax import lax
from jax.experimental import pallas as pl
from jax.experimental.pallas import tpu as pltpu
from jax.experimental.pallas import tpu_sc as plsc

_N = 10000
_E = 320000
_SEQ_DIM = 9216
_NP = 10240
_BN = 512
_GRID = _NP // _BN
_NSUB = 16
_EPT = _E // _NSUB
_STR = _NP // _NSUB
_L = 16


def _gelu(t):
    return 0.5 * t * (1.0 + lax.erf(t * 0.7071067811865476))


def _dense_body(x_ref, pause_ref, seq_ref, encwt_ref, params_ref, sl_ref, srb_ref):
    enc = jnp.dot(seq_ref[...], encwt_ref[...], preferred_element_type=jnp.float32)
    fcw = params_ref[0, :32]
    fcb = params_ref[1, :32]
    encb = params_ref[2, :32]
    wl = params_ref[3, :64]
    wr = params_ref[4, :64]
    linlb = params_ref[5, 0]
    g = _gelu(enc + encb[None, :])
    h = x_ref[...][:, None] * fcw[None, :] + fcb[None, :] + g
    hp = pause_ref[...][:, None] * fcw[None, :] + fcb[None, :]
    fh = _gelu(h)
    fhp = _gelu(hp)
    sl_ref[...] = jnp.sum(fhp * wl[None, :32], axis=1) + jnp.sum(fh * wl[None, 32:], axis=1)
    srb_ref[...] = (jnp.sum(fhp * wr[None, :32], axis=1)
                    + jnp.sum(fh * wr[None, 32:], axis=1) + linlb)


def _dense(x, pause, seq, encwt, params):
    return pl.pallas_call(
        _dense_body,
        grid=(_GRID,),
        in_specs=[
            pl.BlockSpec((_BN,), lambda i: (i,)),
            pl.BlockSpec((_BN,), lambda i: (i,)),
            pl.BlockSpec((_BN, _SEQ_DIM), lambda i: (i, 0)),
            pl.BlockSpec((_SEQ_DIM, 32), lambda i: (0, 0)),
            pl.BlockSpec((8, 128), lambda i: (0, 0)),
        ],
        out_specs=[pl.BlockSpec((_BN,), lambda i: (i,)),
                   pl.BlockSpec((_BN,), lambda i: (i,))],
        out_shape=[jax.ShapeDtypeStruct((_NP,), jnp.float32),
                   jax.ShapeDtypeStruct((_NP,), jnp.float32)],
    )(x, pause, seq, encwt, params)


def _sc_compiler_params():
    cp = pltpu.CompilerParams()
    if "needs_layout_passes" in pltpu.CompilerParams.__dataclass_fields__:
        cp = dataclasses.replace(cp, needs_layout_passes=False)
    return cp


def _edge(sl, srb, src, dst):
    mesh = plsc.VectorSubcoreMesh(core_axis_name="c", subcore_axis_name="s")

    @functools.partial(
        pl.kernel,
        mesh=mesh,
        compiler_params=_sc_compiler_params(),
        out_type=jax.ShapeDtypeStruct((_NP,), jnp.float32),
        scratch_types=[
            pltpu.VMEM((_NP,), jnp.float32),
            pltpu.VMEM((_NP,), jnp.float32),
            pltpu.VMEM((_NP,), jnp.float32),
            pltpu.VMEM((_EPT,), jnp.int32),
            pltpu.VMEM((_EPT,), jnp.int32),
            pltpu.VMEM((_STR,), jnp.float32),
            pltpu.VMEM((_STR,), jnp.float32),
            pltpu.VMEM((_STR,), jnp.float32),
            pltpu.VMEM((_STR,), jnp.float32),
            pltpu.VMEM_SHARED((_NSUB, _NP), jnp.float32),
            pltpu.VMEM_SHARED((_NSUB, _NP), jnp.float32),
        ],
    )
    def k(sl_hbm, srb_hbm, src_hbm, dst_hbm, out_hbm,
          table, acc, cnt, srcb, dstb, sacc, scnt, stmp, ssrb, acc_all, cnt_all):
        cid = lax.axis_index("c")
        sid = lax.axis_index("s")

        @pl.when(cid == 0)
        def _():
            zv = jnp.zeros((_L,), jnp.float32)
            ones = jnp.ones((_L,), jnp.float32)
            pltpu.sync_copy(sl_hbm, table)
            pltpu.sync_copy(src_hbm.at[pl.ds(sid * _EPT, _EPT)], srcb)
            pltpu.sync_copy(dst_hbm.at[pl.ds(sid * _EPT, _EPT)], dstb)

            @pl.loop(0, _NP, step=_L)
            def _(i):
                acc[pl.ds(i, _L)] = zv
                cnt[pl.ds(i, _L)] = zv

            @pl.loop(0, _EPT, step=_L)
            def _(i):
                sv = srcb[pl.ds(i, _L)]
                dv = dstb[pl.ds(i, _L)]
                vals = plsc.load_gather(table, [sv])
                plsc.addupdate_scatter(acc, [dv], vals)
                plsc.addupdate_scatter(cnt, [dv], ones)

            pltpu.sync_copy(acc, acc_all.at[sid])
            pltpu.sync_copy(cnt, cnt_all.at[sid])
            plsc.subcore_barrier()

            base = sid * _STR
            pltpu.sync_copy(acc_all.at[0, pl.ds(base, _STR)], sacc)
            pltpu.sync_copy(cnt_all.at[0, pl.ds(base, _STR)], scnt)

            @pl.loop(1, _NSUB)
            def _(j):
                pltpu.sync_copy(acc_all.at[j, pl.ds(base, _STR)], stmp)

                @pl.loop(0, _STR, step=_L)
                def _(k2):
                    sacc[pl.ds(k2, _L)] = sacc[pl.ds(k2, _L)] + stmp[pl.ds(k2, _L)]

                pltpu.sync_copy(cnt_all.at[j, pl.ds(base, _STR)], stmp)

                @pl.loop(0, _STR, step=_L)
                def _(k2):
                    scnt[pl.ds(k2, _L)] = scnt[pl.ds(k2, _L)] + stmp[pl.ds(k2, _L)]

            pltpu.sync_copy(srb_hbm.at[pl.ds(base, _STR)], ssrb)

            @pl.loop(0, _STR, step=_L)
            def _(k2):
                mean = sacc[pl.ds(k2, _L)] / jnp.maximum(scnt[pl.ds(k2, _L)], 1.0)
                sacc[pl.ds(k2, _L)] = mean + ssrb[pl.ds(k2, _L)]

            pltpu.sync_copy(sacc, out_hbm.at[pl.ds(base, _STR)])

    return k(sl, srb, src, dst)


def kernel(x, seq, pause, edge_index, fc_W, fc_b, enc_W, enc_b, lin_l_W, lin_l_b, lin_r_W):
    f32 = jnp.float32
    src = edge_index[0].astype(jnp.int32)
    dst = edge_index[1].astype(jnp.int32)
    encwt = enc_W.T.astype(f32)
    params = jnp.zeros((8, 128), f32)
    params = params.at[0, :32].set(fc_W[:, 0])
    params = params.at[1, :32].set(fc_b)
    params = params.at[2, :32].set(enc_b)
    params = params.at[3, :64].set(lin_l_W[0])
    params = params.at[4, :64].set(lin_r_W[0])
    params = params.at[5, 0].set(lin_l_b[0])
    sl, srb = _dense(x, pause, seq, encwt, params)
    out = _edge(sl, srb, src, dst)
    return out[:_N].reshape(_N, 1)


# speedup vs baseline: 11.9690x; 1.0037x over previous
import dataclasses
import functools

import jax
import jax.numpy as jnp
from jax import lax
from jax.experimental import pallas as pl
from jax.experimental.pallas import tpu as pltpu
from jax.experimental.pallas import tpu_sc as plsc

_N = 10000
_E = 320000
_SEQ_DIM = 9216
_NP = 10240
_BN = 512
_GRID = _NP // _BN
_NSUB = 16
_EPT = _E // _NSUB
_STR = _NP // _NSUB
_L = 16


def _gelu(t):
    return 0.5 * t * (1.0 + lax.erf(t * 0.7071067811865476))


def _dense_body(x_ref, pause_ref, seq_ref, encwt_ref, params_ref, sl_ref, srb_ref):
    enc = jnp.dot(seq_ref[...].astype(jnp.bfloat16),
                  encwt_ref[...].astype(jnp.bfloat16),
                  preferred_element_type=jnp.float32)
    fcw = params_ref[0, :32]
    fcb = params_ref[1, :32]
    encb = params_ref[2, :32]
    wl = params_ref[3, :64]
    wr = params_ref[4, :64]
    linlb = params_ref[5, 0]
    g = _gelu(enc + encb[None, :])
    h = x_ref[...][:, None] * fcw[None, :] + fcb[None, :] + g
    hp = pause_ref[...][:, None] * fcw[None, :] + fcb[None, :]
    fh = _gelu(h)
    fhp = _gelu(hp)
    sl_ref[...] = jnp.sum(fhp * wl[None, :32], axis=1) + jnp.sum(fh * wl[None, 32:], axis=1)
    srb_ref[...] = (jnp.sum(fhp * wr[None, :32], axis=1)
                    + jnp.sum(fh * wr[None, 32:], axis=1) + linlb)


def _dense(x, pause, seq, encwt, params):
    return pl.pallas_call(
        _dense_body,
        grid=(_GRID,),
        in_specs=[
            pl.BlockSpec((_BN,), lambda i: (i,)),
            pl.BlockSpec((_BN,), lambda i: (i,)),
            pl.BlockSpec((_BN, _SEQ_DIM), lambda i: (i, 0)),
            pl.BlockSpec((_SEQ_DIM, 32), lambda i: (0, 0)),
            pl.BlockSpec((8, 128), lambda i: (0, 0)),
        ],
        out_specs=[pl.BlockSpec((_BN,), lambda i: (i,)),
                   pl.BlockSpec((_BN,), lambda i: (i,))],
        out_shape=[jax.ShapeDtypeStruct((_NP,), jnp.float32),
                   jax.ShapeDtypeStruct((_NP,), jnp.float32)],
    )(x, pause, seq, encwt, params)


def _sc_compiler_params():
    cp = pltpu.CompilerParams()
    if "needs_layout_passes" in pltpu.CompilerParams.__dataclass_fields__:
        cp = dataclasses.replace(cp, needs_layout_passes=False)
    return cp


def _edge(sl, srb, src, dst):
    mesh = plsc.VectorSubcoreMesh(core_axis_name="c", subcore_axis_name="s")

    @functools.partial(
        pl.kernel,
        mesh=mesh,
        compiler_params=_sc_compiler_params(),
        out_type=jax.ShapeDtypeStruct((_NP,), jnp.float32),
        scratch_types=[
            pltpu.VMEM((_NP,), jnp.float32),
            pltpu.VMEM((_NP,), jnp.float32),
            pltpu.VMEM((_NP,), jnp.float32),
            pltpu.VMEM((_EPT,), jnp.int32),
            pltpu.VMEM((_EPT,), jnp.int32),
            pltpu.VMEM((_STR,), jnp.float32),
            pltpu.VMEM((_STR,), jnp.float32),
            pltpu.VMEM((_STR,), jnp.float32),
            pltpu.VMEM((_STR,), jnp.float32),
            pltpu.VMEM_SHARED((_NSUB, _NP), jnp.float32),
            pltpu.VMEM_SHARED((_NSUB, _NP), jnp.float32),
        ],
    )
    def k(sl_hbm, srb_hbm, src_hbm, dst_hbm, out_hbm,
          table, acc, cnt, srcb, dstb, sacc, scnt, stmp, ssrb, acc_all, cnt_all):
        cid = lax.axis_index("c")
        sid = lax.axis_index("s")

        @pl.when(cid == 0)
        def _():
            zv = jnp.zeros((_L,), jnp.float32)
            ones = jnp.ones((_L,), jnp.float32)
            pltpu.sync_copy(sl_hbm, table)
            pltpu.sync_copy(src_hbm.at[pl.ds(sid * _EPT, _EPT)], srcb)
            pltpu.sync_copy(dst_hbm.at[pl.ds(sid * _EPT, _EPT)], dstb)

            @pl.loop(0, _NP, step=_L)
            def _(i):
                acc[pl.ds(i, _L)] = zv
                cnt[pl.ds(i, _L)] = zv

            @pl.loop(0, _EPT, step=_L)
            def _(i):
                sv = srcb[pl.ds(i, _L)]
                dv = dstb[pl.ds(i, _L)]
                vals = plsc.load_gather(table, [sv])
                plsc.addupdate_scatter(acc, [dv], vals)
                plsc.addupdate_scatter(cnt, [dv], ones)

            pltpu.sync_copy(acc, acc_all.at[sid])
            pltpu.sync_copy(cnt, cnt_all.at[sid])
            plsc.subcore_barrier()

            base = sid * _STR
            pltpu.sync_copy(acc_all.at[0, pl.ds(base, _STR)], sacc)
            pltpu.sync_copy(cnt_all.at[0, pl.ds(base, _STR)], scnt)

            @pl.loop(1, _NSUB)
            def _(j):
                pltpu.sync_copy(acc_all.at[j, pl.ds(base, _STR)], stmp)

                @pl.loop(0, _STR, step=_L)
                def _(k2):
                    sacc[pl.ds(k2, _L)] = sacc[pl.ds(k2, _L)] + stmp[pl.ds(k2, _L)]

                pltpu.sync_copy(cnt_all.at[j, pl.ds(base, _STR)], stmp)

                @pl.loop(0, _STR, step=_L)
                def _(k2):
                    scnt[pl.ds(k2, _L)] = scnt[pl.ds(k2, _L)] + stmp[pl.ds(k2, _L)]

            pltpu.sync_copy(srb_hbm.at[pl.ds(base, _STR)], ssrb)

            @pl.loop(0, _STR, step=_L)
            def _(k2):
                mean = sacc[pl.ds(k2, _L)] / jnp.maximum(scnt[pl.ds(k2, _L)], 1.0)
                sacc[pl.ds(k2, _L)] = mean + ssrb[pl.ds(k2, _L)]

            pltpu.sync_copy(sacc, out_hbm.at[pl.ds(base, _STR)])

    return k(sl, srb, src, dst)


def kernel(x, seq, pause, edge_index, fc_W, fc_b, enc_W, enc_b, lin_l_W, lin_l_b, lin_r_W):
    f32 = jnp.float32
    src = edge_index[0].astype(jnp.int32)
    dst = edge_index[1].astype(jnp.int32)
    encwt = enc_W.T.astype(f32)
    params = jnp.zeros((8, 128), f32)
    params = params.at[0, :32].set(fc_W[:, 0])
    params = params.at[1, :32].set(fc_b)
    params = params.at[2, :32].set(enc_b)
    params = params.at[3, :64].set(lin_l_W[0])
    params = params.at[4, :64].set(lin_r_W[0])
    params = params.at[5, 0].set(lin_l_b[0])
    sl, srb = _dense(x, pause, seq, encwt, params)
    out = _edge(sl, srb, src, dst)
    return out[:_N].reshape(_N, 1)
